# baseline (speedup 1.0000x reference)
import functools

import jax
import jax.numpy as jnp
from jax import lax
from jax.experimental import pallas as pl
from jax.experimental.pallas import tpu as pltpu
from jax.experimental.pallas import tpu_sc as plsc

_SCALE = 10.0


@functools.cache
def _make_sc_lookup(B, V, D):
    info = plsc.get_sparse_core_info()
    NC, NS, L = info.num_cores, info.num_subcores, info.num_lanes
    NW = NC * NS
    assert D % NW == 0 and B % L == 0
    rows_per_w = D // NW
    C = 2048
    n_chunks = B // C
    assert B % C == 0
    mesh = plsc.VectorSubcoreMesh(core_axis_name="c", subcore_axis_name="s")

    @functools.partial(
        pl.kernel,
        mesh=mesh,
        out_type=jax.ShapeDtypeStruct((D, B), jnp.float32),
        compiler_params=pltpu.CompilerParams(
            needs_layout_passes=False,
            disable_bounds_checks=True,
            disable_semaphore_checks=True,
        ),
        scratch_types=[
            pltpu.VMEM((V,), jnp.float32),
            pltpu.VMEM((B,), jnp.int32),
            pltpu.VMEM((C,), jnp.float32),
            pltpu.VMEM((C,), jnp.float32),
            pltpu.SemaphoreType.DMA,
            pltpu.SemaphoreType.DMA,
            pltpu.SemaphoreType.DMA,
            pltpu.SemaphoreType.DMA,
        ],
    )
    def lookup(idx_hbm, table_hbm, out_hbm, row_v, idx_v, ob0, ob1, rsem,
               isem, ws0, ws1):
        wid = lax.axis_index("s") * NC + lax.axis_index("c")
        obufs = (ob0, ob1)
        wsems = (ws0, ws1)

        pltpu.async_copy(idx_hbm, idx_v, isem)
        idx_waited = False

        for r in range(rows_per_w):
            d = wid * rows_per_w + r
            pltpu.async_copy(table_hbm.at[d], row_v, rsem)
            pltpu.make_async_copy(table_hbm.at[d], row_v, rsem).wait()
            if not idx_waited:
                pltpu.make_async_copy(idx_hbm, idx_v, isem).wait()
                idx_waited = True
            for c in range(n_chunks):
                buf = obufs[c % 2]
                sem = wsems[c % 2]
                if r > 0 or c >= 2:
                    pltpu.make_async_copy(
                        out_hbm.at[d, pl.ds(c * C, C)], buf, sem
                    ).wait()

                @plsc.parallel_loop(0, C // L, unroll=16)
                def gather16(k):
                    sl = pl.ds(k * L, L)
                    iv = idx_v[pl.ds(c * C + k * L, L)]
                    buf[sl] = plsc.load_gather(row_v, [iv]) * _SCALE

                pltpu.async_copy(
                    buf, out_hbm.at[d, pl.ds(c * C, C)], sem
                )
        pltpu.make_async_copy(out_hbm.at[0, pl.ds(0, C)], ob0, ws0).wait()
        pltpu.make_async_copy(out_hbm.at[0, pl.ds(0, C)], ob1, ws1).wait()

    return lookup


def kernel(x, weight):
    (B,) = x.shape
    V, D = weight.shape
    fn = _make_sc_lookup(B, V, D)
    outT = fn(x.astype(jnp.int32), weight.T)
    return outT.T


# speedup vs baseline: 1.0412x; 1.0412x over previous
import functools

import jax
import jax.numpy as jnp
from jax import lax
from jax.experimental import pallas as pl
from jax.experimental.pallas import tpu as pltpu
from jax.experimental.pallas import tpu_sc as plsc

_SCALE = 10.0


@functools.cache
def _make_sc_lookup(B, V, D):
    info = plsc.get_sparse_core_info()
    NC, NS, L = info.num_cores, info.num_subcores, info.num_lanes
    NW = NC * NS
    assert D % NW == 0 and B % L == 0
    rows_per_w = D // NW
    C = 4096
    n_chunks = B // C
    assert B % C == 0
    mesh = plsc.VectorSubcoreMesh(core_axis_name="c", subcore_axis_name="s")

    @functools.partial(
        pl.kernel,
        mesh=mesh,
        out_type=jax.ShapeDtypeStruct((D, B), jnp.float32),
        compiler_params=pltpu.CompilerParams(
            needs_layout_passes=False,
            disable_bounds_checks=True,
            disable_semaphore_checks=True,
        ),
        scratch_types=[
            pltpu.VMEM((V,), jnp.float32),
            pltpu.VMEM((B,), jnp.int32),
            pltpu.VMEM((C,), jnp.float32),
            pltpu.VMEM((C,), jnp.float32),
            pltpu.SemaphoreType.DMA,
            pltpu.SemaphoreType.DMA,
            pltpu.SemaphoreType.DMA,
            pltpu.SemaphoreType.DMA,
        ],
    )
    def lookup(idx_hbm, table_hbm, out_hbm, row_v, idx_v, ob0, ob1, rsem,
               isem, ws0, ws1):
        wid = lax.axis_index("s") * NC + lax.axis_index("c")
        obufs = (ob0, ob1)
        wsems = (ws0, ws1)

        pltpu.async_copy(idx_hbm, idx_v, isem)
        idx_waited = False

        for r in range(rows_per_w):
            d = wid * rows_per_w + r
            pltpu.async_copy(table_hbm.at[d], row_v, rsem)
            pltpu.make_async_copy(table_hbm.at[d], row_v, rsem).wait()
            if not idx_waited:
                pltpu.make_async_copy(idx_hbm, idx_v, isem).wait()
                idx_waited = True
            for c in range(n_chunks):
                buf = obufs[c % 2]
                sem = wsems[c % 2]
                if r > 0 or c >= 2:
                    pltpu.make_async_copy(
                        out_hbm.at[d, pl.ds(c * C, C)], buf, sem
                    ).wait()

                @plsc.parallel_loop(0, C // L, unroll=8)
                def gather16(k):
                    sl = pl.ds(k * L, L)
                    iv = idx_v[pl.ds(c * C + k * L, L)]
                    buf[sl] = plsc.load_gather(row_v, [iv]) * _SCALE

                pltpu.async_copy(
                    buf, out_hbm.at[d, pl.ds(c * C, C)], sem
                )
        pltpu.make_async_copy(out_hbm.at[0, pl.ds(0, C)], ob0, ws0).wait()
        pltpu.make_async_copy(out_hbm.at[0, pl.ds(0, C)], ob1, ws1).wait()

    return lookup


def kernel(x, weight):
    (B,) = x.shape
    V, D = weight.shape
    fn = _make_sc_lookup(B, V, D)
    outT = fn(x.astype(jnp.int32), weight.T)
    return outT.T
